# Initial kernel scaffold; baseline (speedup 1.0000x reference)
import jax
import jax.numpy as jnp
from jax.experimental import pallas as pl


def kernel(x, table, W1, b1, W2, b2):
    raise NotImplementedError("write your pallas kernel here")


# speedup vs baseline: 7.6250x; 7.6250x over previous
import functools

import jax
import jax.numpy as jnp
from jax import lax
from jax.experimental import pallas as pl
from jax.experimental.pallas import tpu as pltpu
from jax.experimental.pallas import tpu_sc as plsc

VOCAB = 1000000
EMBED = 16
BATCH = 16384
HIST = 200

NC = 2
NS = 16
L = 16
NW = NC * NS
B_PER_W = BATCH // NW
CHUNK_ROWS = 16
CHUNK_IDX = CHUNK_ROWS * HIST
N_CHUNKS = B_PER_W // CHUNK_ROWS


def _make_sc_kernel():
  mesh = plsc.VectorSubcoreMesh(core_axis_name="c", subcore_axis_name="s")

  @functools.partial(
      pl.kernel,
      mesh=mesh,
      compiler_params=pltpu.CompilerParams(use_tc_tiling_on_sc=False),
      out_type=jax.ShapeDtypeStruct((BATCH, EMBED), jnp.float32),
      scratch_types=[
          pltpu.VMEM((CHUNK_IDX,), jnp.int32),
          pltpu.VMEM((CHUNK_IDX, EMBED), jnp.float32),
          pltpu.VMEM((CHUNK_ROWS, EMBED), jnp.float32),
          pltpu.SemaphoreType.DMA,
      ],
  )
  def sc_embed_sum(xflat, table, out, idx_v, rows_v, acc_v, sem):
    wid = lax.axis_index("s") * NC + lax.axis_index("c")
    base = wid * B_PER_W

    def chunk_body(ci, carry):
      row0 = base + ci * CHUNK_ROWS
      pltpu.sync_copy(xflat.at[pl.ds(row0 * HIST, CHUNK_IDX)], idx_v)
      pltpu.async_copy(table.at[idx_v], rows_v, sem).wait()
      for r in range(CHUNK_ROWS):
        def l_body(li, acc, r=r):
          return acc + rows_v[r * HIST + li]
        acc = lax.fori_loop(0, HIST, l_body, jnp.zeros((L,), jnp.float32),
                            unroll=8)
        acc_v[r] = acc
      pltpu.sync_copy(acc_v, out.at[pl.ds(row0, CHUNK_ROWS), :])
      return carry

    lax.fori_loop(0, N_CHUNKS, chunk_body, 0)

  return sc_embed_sum


_SC_EMBED_SUM = _make_sc_kernel()

_TC_BLOCK = 4096


def _tc_head_body(sums_ref, v_ref, c_ref, out_ref):
  z = jnp.sum(sums_ref[...] * v_ref[...], axis=1, keepdims=True)
  z = z * jnp.float32(1.0 / HIST) + c_ref[0, 0]
  out_ref[...] = 1.0 / (1.0 + jnp.exp(-z))


def _tc_head(sums, v, c):
  grid = BATCH // _TC_BLOCK
  return pl.pallas_call(
      _tc_head_body,
      grid=(grid,),
      in_specs=[
          pl.BlockSpec((_TC_BLOCK, EMBED), lambda i: (i, 0)),
          pl.BlockSpec((1, EMBED), lambda i: (0, 0)),
          pl.BlockSpec(memory_space=pltpu.SMEM),
      ],
      out_specs=pl.BlockSpec((_TC_BLOCK, 1), lambda i: (i, 0)),
      out_shape=jax.ShapeDtypeStruct((BATCH, 1), jnp.float32),
  )(sums, v, c)


def kernel(x, table, W1, b1, W2, b2):
  v = (W2 @ W1).reshape(1, EMBED)
  c = (W2 @ b1 + b2).reshape(1, 1)
  sums = _SC_EMBED_SUM(x.reshape(-1), table)
  return _tc_head(sums, v.astype(jnp.float32), c.astype(jnp.float32))
